# baseline (speedup 1.0000x reference)
import functools

import jax
import jax.numpy as jnp
from jax.experimental import pallas as pl
from jax.experimental.pallas import tpu as pltpu

N_EXPERTS = 64
BLOCK_ROWS = 512


def _router_block(x_ref, wt_ref, b_ref, scores_ref, tv_ref, ti_ref):
    logits = jnp.dot(x_ref[...], wt_ref[...],
                     preferred_element_type=jnp.float32) + b_ref[...]
    m = jnp.max(logits, axis=-1, keepdims=True)
    e = jnp.exp(logits - m)
    s = jnp.sum(e, axis=-1, keepdims=True)
    sc = e / s
    scores_ref[...] = sc

    idx = jax.lax.broadcasted_iota(jnp.int32, sc.shape, 1)
    v1 = jnp.max(sc, axis=-1, keepdims=True)
    i1 = jnp.argmax(sc, axis=-1)[:, None]
    masked = jnp.where(idx == i1, -1.0, sc)
    v2 = jnp.max(masked, axis=-1, keepdims=True)
    i2 = jnp.argmax(masked, axis=-1)[:, None]

    tv_ref[...] = jnp.concatenate([v1, v2], axis=-1)
    ti_ref[...] = jnp.concatenate([i1, i2], axis=-1)


@jax.jit
def kernel(x, W, b):
    n, d = x.shape
    e = W.shape[0]
    wt = W.T
    b2 = b.reshape(1, e)
    grid = (n // BLOCK_ROWS,)
    scores, tv, ti = pl.pallas_call(
        _router_block,
        grid=grid,
        in_specs=[
            pl.BlockSpec((BLOCK_ROWS, d), lambda i: (i, 0)),
            pl.BlockSpec((d, e), lambda i: (0, 0)),
            pl.BlockSpec((1, e), lambda i: (0, 0)),
        ],
        out_specs=[
            pl.BlockSpec((BLOCK_ROWS, e), lambda i: (i, 0)),
            pl.BlockSpec((BLOCK_ROWS, 2), lambda i: (i, 0)),
            pl.BlockSpec((BLOCK_ROWS, 2), lambda i: (i, 0)),
        ],
        out_shape=[
            jax.ShapeDtypeStruct((n, e), jnp.float32),
            jax.ShapeDtypeStruct((n, 2), jnp.float32),
            jax.ShapeDtypeStruct((n, 2), jnp.int32),
        ],
        compiler_params=pltpu.CompilerParams(
            dimension_semantics=("parallel",)),
    )(x, wt, b2)
    return tv, ti, scores


# speedup vs baseline: 1.3596x; 1.3596x over previous
import functools

import jax
import jax.numpy as jnp
from jax.experimental import pallas as pl
from jax.experimental.pallas import tpu as pltpu

N_EXPERTS = 64
BLOCK_ROWS = 2048


def _router_block(x_ref, wt_ref, b_ref, scores_ref, tv_ref, ti_ref):
    logits = jnp.dot(x_ref[...], wt_ref[...],
                     preferred_element_type=jnp.float32) + b_ref[...]
    m = jnp.max(logits, axis=-1, keepdims=True)
    e = jnp.exp(logits - m)
    s = jnp.sum(e, axis=-1, keepdims=True)
    sc = e / s
    scores_ref[...] = sc

    idx = jax.lax.broadcasted_iota(jnp.int32, sc.shape, 1)
    v1 = jnp.max(sc, axis=-1, keepdims=True)
    i1 = jnp.argmax(sc, axis=-1)[:, None]
    masked = jnp.where(idx == i1, -1.0, sc)
    v2 = jnp.max(masked, axis=-1, keepdims=True)
    i2 = jnp.argmax(masked, axis=-1)[:, None]

    tv_ref[...] = jnp.concatenate([v1, v2], axis=-1)
    ti_ref[...] = jnp.concatenate([i1, i2], axis=-1)


@jax.jit
def kernel(x, W, b):
    n, d = x.shape
    e = W.shape[0]
    wt = W.T
    b2 = b.reshape(1, e)
    grid = (n // BLOCK_ROWS,)
    scores, tv, ti = pl.pallas_call(
        _router_block,
        grid=grid,
        in_specs=[
            pl.BlockSpec((BLOCK_ROWS, d), lambda i: (i, 0)),
            pl.BlockSpec((d, e), lambda i: (0, 0)),
            pl.BlockSpec((1, e), lambda i: (0, 0)),
        ],
        out_specs=[
            pl.BlockSpec((BLOCK_ROWS, e), lambda i: (i, 0)),
            pl.BlockSpec((BLOCK_ROWS, 2), lambda i: (i, 0)),
            pl.BlockSpec((BLOCK_ROWS, 2), lambda i: (i, 0)),
        ],
        out_shape=[
            jax.ShapeDtypeStruct((n, e), jnp.float32),
            jax.ShapeDtypeStruct((n, 2), jnp.float32),
            jax.ShapeDtypeStruct((n, 2), jnp.int32),
        ],
        compiler_params=pltpu.CompilerParams(
            dimension_semantics=("parallel",)),
    )(x, wt, b2)
    return tv, ti, scores


# speedup vs baseline: 1.4326x; 1.0537x over previous
import functools

import jax
import jax.numpy as jnp
from jax.experimental import pallas as pl
from jax.experimental.pallas import tpu as pltpu

N_EXPERTS = 64
BLOCK_ROWS = 4096


def _router_block(x_ref, wt_ref, b_ref, scores_ref, tv_ref, ti_ref):
    logits = jnp.dot(x_ref[...], wt_ref[...],
                     preferred_element_type=jnp.float32) + b_ref[...]
    m = jnp.max(logits, axis=-1, keepdims=True)
    e = jnp.exp(logits - m)
    s = jnp.sum(e, axis=-1, keepdims=True)
    sc = e / s
    scores_ref[...] = sc

    idx = jax.lax.broadcasted_iota(jnp.int32, sc.shape, 1)
    v1 = jnp.max(sc, axis=-1, keepdims=True)
    i1 = jnp.argmax(sc, axis=-1)[:, None]
    masked = jnp.where(idx == i1, -1.0, sc)
    v2 = jnp.max(masked, axis=-1, keepdims=True)
    i2 = jnp.argmax(masked, axis=-1)[:, None]

    tv_ref[...] = jnp.concatenate([v1, v2], axis=-1)
    ti_ref[...] = jnp.concatenate([i1, i2], axis=-1)


@jax.jit
def kernel(x, W, b):
    n, d = x.shape
    e = W.shape[0]
    wt = W.T
    b2 = b.reshape(1, e)
    grid = (n // BLOCK_ROWS,)
    scores, tv, ti = pl.pallas_call(
        _router_block,
        grid=grid,
        in_specs=[
            pl.BlockSpec((BLOCK_ROWS, d), lambda i: (i, 0)),
            pl.BlockSpec((d, e), lambda i: (0, 0)),
            pl.BlockSpec((1, e), lambda i: (0, 0)),
        ],
        out_specs=[
            pl.BlockSpec((BLOCK_ROWS, e), lambda i: (i, 0)),
            pl.BlockSpec((BLOCK_ROWS, 2), lambda i: (i, 0)),
            pl.BlockSpec((BLOCK_ROWS, 2), lambda i: (i, 0)),
        ],
        out_shape=[
            jax.ShapeDtypeStruct((n, e), jnp.float32),
            jax.ShapeDtypeStruct((n, 2), jnp.float32),
            jax.ShapeDtypeStruct((n, 2), jnp.int32),
        ],
        compiler_params=pltpu.CompilerParams(
            dimension_semantics=("parallel",)),
    )(x, wt, b2)
    return tv, ti, scores


# speedup vs baseline: 1.4423x; 1.0068x over previous
import jax
import jax.numpy as jnp
from jax.experimental import pallas as pl
from jax.experimental.pallas import tpu as pltpu

N_EXPERTS = 64
BLOCK_ROWS = 4096
HALF_D = 384


def _router_block(xa_ref, xb_ref, wt_ref, b_ref, scores_ref, tv_ref, ti_ref):
    logits = (jnp.dot(xa_ref[...], wt_ref[:HALF_D, :],
                      preferred_element_type=jnp.float32)
              + jnp.dot(xb_ref[...], wt_ref[HALF_D:, :],
                        preferred_element_type=jnp.float32)
              + b_ref[...])
    m = jnp.max(logits, axis=-1, keepdims=True)
    e = jnp.exp(logits - m)
    s = jnp.sum(e, axis=-1, keepdims=True)
    sc = e / s
    scores_ref[...] = sc

    idx = jax.lax.broadcasted_iota(jnp.int32, sc.shape, 1)
    v1 = jnp.max(sc, axis=-1, keepdims=True)
    i1 = jnp.argmax(sc, axis=-1)[:, None]
    masked = jnp.where(idx == i1, -1.0, sc)
    v2 = jnp.max(masked, axis=-1, keepdims=True)
    i2 = jnp.argmax(masked, axis=-1)[:, None]

    tv_ref[...] = jnp.concatenate([v1, v2], axis=-1)
    ti_ref[...] = jnp.concatenate([i1, i2], axis=-1)


@jax.jit
def kernel(x, W, b):
    n, d = x.shape
    e = W.shape[0]
    wt = W.T
    b2 = b.reshape(1, e)
    grid = (n // BLOCK_ROWS,)
    scores, tv, ti = pl.pallas_call(
        _router_block,
        grid=grid,
        in_specs=[
            pl.BlockSpec((BLOCK_ROWS, HALF_D), lambda i: (i, 0)),
            pl.BlockSpec((BLOCK_ROWS, HALF_D), lambda i: (i, 1)),
            pl.BlockSpec((d, e), lambda i: (0, 0)),
            pl.BlockSpec((1, e), lambda i: (0, 0)),
        ],
        out_specs=[
            pl.BlockSpec((BLOCK_ROWS, e), lambda i: (i, 0)),
            pl.BlockSpec((BLOCK_ROWS, 2), lambda i: (i, 0)),
            pl.BlockSpec((BLOCK_ROWS, 2), lambda i: (i, 0)),
        ],
        out_shape=[
            jax.ShapeDtypeStruct((n, e), jnp.float32),
            jax.ShapeDtypeStruct((n, 2), jnp.float32),
            jax.ShapeDtypeStruct((n, 2), jnp.int32),
        ],
        compiler_params=pltpu.CompilerParams(
            dimension_semantics=("parallel",)),
    )(x, x, wt, b2)
    return tv, ti, scores
